# baseline (speedup 1.0000x reference)
import functools

import jax
import jax.numpy as jnp
from jax import lax
from jax.experimental import pallas as pl
from jax.experimental.pallas import tpu as pltpu
from jax.experimental.pallas import tpu_sc as plsc

_NC = 2
_NS = 16
_NW = _NC * _NS
_L = 16


@functools.lru_cache(maxsize=None)
def _build(F: int, B_TOK: int, V: int, D: int):
    assert D == 2 * _L and B_TOK % 128 == 0
    NBLK = F * (B_TOK // 128)
    assert NBLK % _NW == 0
    ITERS = NBLK // _NW

    mesh = plsc.VectorSubcoreMesh(core_axis_name="c", subcore_axis_name="s")

    @functools.partial(
        pl.kernel,
        mesh=mesh,
        out_type=jax.ShapeDtypeStruct((F, D, B_TOK), jnp.float32),
        scratch_types=[
            pltpu.VMEM((128,), jnp.int32),
            pltpu.VMEM((128, D), jnp.float32),
            pltpu.VMEM((D, 128), jnp.float32),
            pltpu.SemaphoreType.DMA,
        ],
        compiler_params=pltpu.CompilerParams(
            use_tc_tiling_on_sc=False, needs_layout_passes=False),
    )
    def emb_kernel(xT_hbm, w_hbm, out_hbm, idx_v, gbuf, obuf, gsem):
        c = lax.axis_index("c")
        s = lax.axis_index("s")
        w = s * _NC + c
        lane = lax.iota(jnp.int32, _L)

        def body(i, carry):
            bid = i * _NW + w
            f = bid // (B_TOK // 128)
            cb = bid % (B_TOK // 128)
            pltpu.sync_copy(xT_hbm.at[f, pl.ds(cb * 128, 128)], idx_v)
            pltpu.async_copy(w_hbm.at[idx_v], gbuf, gsem).wait()
            for t in range(128):
                for h in range(2):
                    v = gbuf[t, pl.ds(h * _L, _L)]
                    plsc.store_scatter(
                        obuf,
                        [lane + (h * _L), jnp.full((_L,), t, jnp.int32)], v)
            pltpu.sync_copy(obuf,
                            out_hbm.at[f, :, pl.ds(cb * 128, 128)])
            return carry

        lax.fori_loop(0, ITERS, body, 0)

    return emb_kernel


def kernel(x, weight):
    B_TOK, F = x.shape
    V, D = weight.shape
    xT = x.astype(jnp.int32).T
    out3 = _build(F, B_TOK, V, D)(xT, weight)
    return jnp.transpose(out3, (2, 0, 1))


# speedup vs baseline: 1.2035x; 1.2035x over previous
import functools

import jax
import jax.numpy as jnp
from jax import lax
from jax.experimental import pallas as pl
from jax.experimental.pallas import tpu as pltpu
from jax.experimental.pallas import tpu_sc as plsc

_NC = 2
_NS = 16
_NW = _NC * _NS
_L = 16


@functools.lru_cache(maxsize=None)
def _build(F: int, B_TOK: int, V: int, D: int):
    assert D == 2 * _L and B_TOK % 128 == 0
    NBLK = F * (B_TOK // 128)
    assert NBLK % _NW == 0
    ITERS = NBLK // _NW

    mesh = plsc.VectorSubcoreMesh(core_axis_name="c", subcore_axis_name="s")

    @functools.partial(
        pl.kernel,
        mesh=mesh,
        out_type=jax.ShapeDtypeStruct((F, D, B_TOK), jnp.float32),
        scratch_types=[
            pltpu.VMEM((128,), jnp.int32),
            pltpu.VMEM((128, D), jnp.float32),
            pltpu.VMEM((D, 129), jnp.float32),
            pltpu.SemaphoreType.DMA,
        ],
        compiler_params=pltpu.CompilerParams(
            use_tc_tiling_on_sc=False, needs_layout_passes=False),
    )
    def emb_kernel(xT_hbm, w_hbm, out_hbm, idx_v, gbuf, obuf, gsem):
        c = lax.axis_index("c")
        s = lax.axis_index("s")
        w = s * _NC + c
        lane = lax.iota(jnp.int32, _L)

        def body(i, carry):
            bid = i * _NW + w
            f = bid // (B_TOK // 128)
            cb = bid % (B_TOK // 128)
            pltpu.sync_copy(xT_hbm.at[f, pl.ds(cb * 128, 128)], idx_v)
            pltpu.async_copy(w_hbm.at[idx_v], gbuf, gsem).wait()
            for t in range(128):
                for h in range(2):
                    v = gbuf[t, pl.ds(h * _L, _L)]
                    plsc.store_scatter(
                        obuf,
                        [lane + (h * _L), jnp.full((_L,), t, jnp.int32)], v)
            pltpu.sync_copy(obuf.at[:, pl.ds(0, 128)],
                            out_hbm.at[f, :, pl.ds(cb * 128, 128)])
            return carry

        lax.fori_loop(0, ITERS, body, 0)

    return emb_kernel


def kernel(x, weight):
    B_TOK, F = x.shape
    V, D = weight.shape
    xT = x.astype(jnp.int32).T
    out3 = _build(F, B_TOK, V, D)(xT, weight)
    return jnp.transpose(out3, (2, 0, 1))


# speedup vs baseline: 1.3290x; 1.1043x over previous
import functools

import jax
import jax.numpy as jnp
from jax import lax
from jax.experimental import pallas as pl
from jax.experimental.pallas import tpu as pltpu
from jax.experimental.pallas import tpu_sc as plsc

_NC = 2
_NS = 16
_NW = _NC * _NS
_L = 16


@functools.lru_cache(maxsize=None)
def _build(F: int, B_TOK: int, V: int, D: int):
    assert D == 2 * _L and B_TOK % 128 == 0
    NBLK = F * (B_TOK // 128)
    assert NBLK % _NW == 0
    ITERS = NBLK // _NW

    mesh = plsc.VectorSubcoreMesh(core_axis_name="c", subcore_axis_name="s")

    @functools.partial(
        pl.kernel,
        mesh=mesh,
        out_type=jax.ShapeDtypeStruct((F, D, B_TOK), jnp.float32),
        scratch_types=[
            pltpu.VMEM((128,), jnp.int32),
            pltpu.VMEM((128,), jnp.int32),
            pltpu.VMEM((128, D), jnp.float32),
            pltpu.VMEM((128, D), jnp.float32),
            pltpu.VMEM((D, 129), jnp.float32),
            pltpu.VMEM((D, 129), jnp.float32),
            pltpu.SemaphoreType.DMA,
            pltpu.SemaphoreType.DMA,
            pltpu.SemaphoreType.DMA,
            pltpu.SemaphoreType.DMA,
            pltpu.SemaphoreType.DMA,
            pltpu.SemaphoreType.DMA,
        ],
        compiler_params=pltpu.CompilerParams(
            use_tc_tiling_on_sc=False, needs_layout_passes=False),
    )
    def emb_kernel(xT_hbm, w_hbm, out_hbm, idx0, idx1, gb0, gb1, ob0, ob1,
                   is0, is1, gs0, gs1, ws0, ws1):
        c = lax.axis_index("c")
        s = lax.axis_index("s")
        w = s * _NC + c
        lane = lax.iota(jnp.int32, _L)
        idx = (idx0, idx1)
        gb = (gb0, gb1)
        ob = (ob0, ob1)
        isem = (is0, is1)
        gsem = (gs0, gs1)
        wsem = (ws0, ws1)

        def fc(i):
            bid = i * _NW + w
            return bid // (B_TOK // 128), bid % (B_TOK // 128)

        def stage_idx(i, p):
            f, cb = fc(i)
            return pltpu.async_copy(
                xT_hbm.at[f, pl.ds(cb * 128, 128)], idx[p], isem[p])

        def fire_gather(p):
            return pltpu.async_copy(w_hbm.at[idx[p]], gb[p], gsem[p])

        def transpose(p):
            for t in range(128):
                for h in range(2):
                    v = gb[p][t, pl.ds(h * _L, _L)]
                    plsc.store_scatter(
                        ob[p],
                        [lane + (h * _L), jnp.full((_L,), t, jnp.int32)], v)

        def fire_write(i, p):
            f, cb = fc(i)
            return pltpu.async_copy(
                ob[p].at[:, pl.ds(0, 128)],
                out_hbm.at[f, :, pl.ds(cb * 128, 128)], wsem[p])

        stage_idx(0, 0).wait()
        fire_gather(0)

        def body(j2, carry):
            for p in range(2):
                i = j2 * 2 + p
                q = 1 - p
                @pl.when(i + 1 < ITERS)
                def _():
                    stage_idx(i + 1, q)
                pltpu.make_async_copy(w_hbm.at[idx[p]], gb[p],
                                      gsem[p]).wait()

                @pl.when(i + 1 < ITERS)
                def _():
                    pltpu.make_async_copy(
                        xT_hbm.at[0, pl.ds(0, 128)], idx[q],
                        isem[q]).wait()
                    fire_gather(q)

                @pl.when(i >= 2)
                def _():
                    f2, cb2 = fc(i - 2)
                    pltpu.make_async_copy(
                        ob[p].at[:, pl.ds(0, 128)],
                        out_hbm.at[f2, :, pl.ds(cb2 * 128, 128)],
                        wsem[p]).wait()
                transpose(p)
                fire_write(i, p)
            return carry

        lax.fori_loop(0, ITERS // 2, body, 0)

        f1, cb1 = fc(ITERS - 1)
        pltpu.make_async_copy(
            ob[(ITERS - 1) % 2].at[:, pl.ds(0, 128)],
            out_hbm.at[f1, :, pl.ds(cb1 * 128, 128)],
            wsem[(ITERS - 1) % 2]).wait()
        f2, cb2 = fc(ITERS - 2)
        pltpu.make_async_copy(
            ob[(ITERS - 2) % 2].at[:, pl.ds(0, 128)],
            out_hbm.at[f2, :, pl.ds(cb2 * 128, 128)],
            wsem[(ITERS - 2) % 2]).wait()

    return emb_kernel


def kernel(x, weight):
    B_TOK, F = x.shape
    V, D = weight.shape
    xT = x.astype(jnp.int32).T
    out3 = _build(F, B_TOK, V, D)(xT, weight)
    return jnp.transpose(out3, (2, 0, 1))


# speedup vs baseline: 1.3417x; 1.0096x over previous
import functools

import jax
import jax.numpy as jnp
from jax import lax
from jax.experimental import pallas as pl
from jax.experimental.pallas import tpu as pltpu
from jax.experimental.pallas import tpu_sc as plsc

_NC = 2
_NS = 16
_NW = _NC * _NS
_L = 16


@functools.lru_cache(maxsize=None)
def _build(F: int, B_TOK: int, V: int, D: int):
    assert D == 2 * _L and B_TOK % 128 == 0
    NBLK = F * (B_TOK // 128)
    assert NBLK % _NW == 0
    ITERS = NBLK // _NW

    mesh = plsc.VectorSubcoreMesh(core_axis_name="c", subcore_axis_name="s")

    @functools.partial(
        pl.kernel,
        mesh=mesh,
        out_type=jax.ShapeDtypeStruct((F, D, B_TOK), jnp.float32),
        scratch_types=[
            pltpu.VMEM((128,), jnp.int32),
            pltpu.VMEM((128,), jnp.int32),
            pltpu.VMEM((128, 128), jnp.float32),
            pltpu.VMEM((128, 128), jnp.float32),
            pltpu.VMEM((D, 129), jnp.float32),
            pltpu.VMEM((D, 129), jnp.float32),
            pltpu.SemaphoreType.DMA,
            pltpu.SemaphoreType.DMA,
            pltpu.SemaphoreType.DMA,
            pltpu.SemaphoreType.DMA,
            pltpu.SemaphoreType.DMA,
            pltpu.SemaphoreType.DMA,
        ],
        compiler_params=pltpu.CompilerParams(
            use_tc_tiling_on_sc=False, needs_layout_passes=False),
    )
    def emb_kernel(xT_hbm, w_hbm, out_hbm, idx0, idx1, gb0, gb1, ob0, ob1,
                   is0, is1, gs0, gs1, ws0, ws1):
        c = lax.axis_index("c")
        s = lax.axis_index("s")
        w = s * _NC + c
        lane = lax.iota(jnp.int32, _L)
        idx = (idx0, idx1)
        gb = (gb0, gb1)
        ob = (ob0, ob1)
        isem = (is0, is1)
        gsem = (gs0, gs1)
        wsem = (ws0, ws1)

        def fc(i):
            bid = i * _NW + w
            return bid // (B_TOK // 128), bid % (B_TOK // 128)

        def stage_idx(i, p):
            f, cb = fc(i)
            return pltpu.async_copy(
                xT_hbm.at[f, pl.ds(cb * 128, 128)], idx[p], isem[p])

        def fire_gather(p):
            return pltpu.async_copy(w_hbm.at[idx[p]], gb[p], gsem[p])

        def transpose(p):
            for t in range(128):
                for h in range(2):
                    v = gb[p][t, pl.ds(h * _L, _L)]
                    plsc.store_scatter(
                        ob[p],
                        [lane + (h * _L), jnp.full((_L,), t, jnp.int32)], v)

        def fire_write(i, p):
            f, cb = fc(i)
            return pltpu.async_copy(
                ob[p].at[:, pl.ds(0, 128)],
                out_hbm.at[f, :, pl.ds(cb * 128, 128)], wsem[p])

        stage_idx(0, 0).wait()
        fire_gather(0)

        def body(j2, carry):
            for p in range(2):
                i = j2 * 2 + p
                q = 1 - p
                @pl.when(i + 1 < ITERS)
                def _():
                    stage_idx(i + 1, q)
                pltpu.make_async_copy(w_hbm.at[idx[p]], gb[p],
                                      gsem[p]).wait()

                @pl.when(i + 1 < ITERS)
                def _():
                    pltpu.make_async_copy(
                        xT_hbm.at[0, pl.ds(0, 128)], idx[q],
                        isem[q]).wait()
                    fire_gather(q)

                @pl.when(i >= 2)
                def _():
                    f2, cb2 = fc(i - 2)
                    pltpu.make_async_copy(
                        ob[p].at[:, pl.ds(0, 128)],
                        out_hbm.at[f2, :, pl.ds(cb2 * 128, 128)],
                        wsem[p]).wait()
                transpose(p)
                fire_write(i, p)
            return carry

        lax.fori_loop(0, ITERS // 2, body, 0)

        f1, cb1 = fc(ITERS - 1)
        pltpu.make_async_copy(
            ob[(ITERS - 1) % 2].at[:, pl.ds(0, 128)],
            out_hbm.at[f1, :, pl.ds(cb1 * 128, 128)],
            wsem[(ITERS - 1) % 2]).wait()
        f2, cb2 = fc(ITERS - 2)
        pltpu.make_async_copy(
            ob[(ITERS - 2) % 2].at[:, pl.ds(0, 128)],
            out_hbm.at[f2, :, pl.ds(cb2 * 128, 128)],
            wsem[(ITERS - 2) % 2]).wait()

    return emb_kernel


def kernel(x, weight):
    B_TOK, F = x.shape
    V, D = weight.shape
    xT = x.astype(jnp.int32).T
    w128 = jnp.pad(weight, ((0, 0), (0, 128 - D)))
    out3 = _build(F, B_TOK, V, D)(xT, w128)
    return jnp.transpose(out3, (2, 0, 1))


# speedup vs baseline: 1.3428x; 1.0008x over previous
import functools

import jax
import jax.numpy as jnp
from jax import lax
from jax.experimental import pallas as pl
from jax.experimental.pallas import tpu as pltpu
from jax.experimental.pallas import tpu_sc as plsc

_NC = 2
_NS = 16
_NW = _NC * _NS
_L = 16


@functools.lru_cache(maxsize=None)
def _build(F: int, B_TOK: int, V: int, D: int):
    assert D == 2 * _L and B_TOK % 128 == 0
    NBLK = F * (B_TOK // 128)
    assert NBLK % _NW == 0
    ITERS = NBLK // _NW
    assert ITERS % 2 == 0 and ITERS >= 4

    mesh = plsc.VectorSubcoreMesh(core_axis_name="c", subcore_axis_name="s")

    @functools.partial(
        pl.kernel,
        mesh=mesh,
        out_type=jax.ShapeDtypeStruct((F, D, B_TOK), jnp.float32),
        scratch_types=[
            pltpu.VMEM((128,), jnp.int32),
            pltpu.VMEM((128,), jnp.int32),
            pltpu.VMEM((128, 128), jnp.float32),
            pltpu.VMEM((128, 128), jnp.float32),
            pltpu.VMEM((D, 129), jnp.float32),
            pltpu.VMEM((D, 129), jnp.float32),
            pltpu.SemaphoreType.DMA,
            pltpu.SemaphoreType.DMA,
            pltpu.SemaphoreType.DMA,
            pltpu.SemaphoreType.DMA,
            pltpu.SemaphoreType.DMA,
            pltpu.SemaphoreType.DMA,
        ],
        compiler_params=pltpu.CompilerParams(
            use_tc_tiling_on_sc=False, needs_layout_passes=False),
    )
    def emb_kernel(xT_hbm, w_hbm, out_hbm, idx0, idx1, gb0, gb1, ob0, ob1,
                   is0, is1, gs0, gs1, ws0, ws1):
        c = lax.axis_index("c")
        s = lax.axis_index("s")
        w = s * _NC + c
        lane = lax.iota(jnp.int32, _L)
        idx = (idx0, idx1)
        gb = (gb0, gb1)
        ob = (ob0, ob1)
        isem = (is0, is1)
        gsem = (gs0, gs1)
        wsem = (ws0, ws1)

        def fc(i):
            bid = i * _NW + w
            return bid // (B_TOK // 128), bid % (B_TOK // 128)

        def stage_idx(i, p):
            f, cb = fc(i)
            return pltpu.async_copy(
                xT_hbm.at[f, pl.ds(cb * 128, 128)], idx[p], isem[p])

        def fire_gather(p):
            return pltpu.async_copy(w_hbm.at[idx[p]], gb[p], gsem[p])

        def transpose(p):
            for t in range(128):
                for h in range(2):
                    v = gb[p][t, pl.ds(h * _L, _L)]
                    plsc.store_scatter(
                        ob[p],
                        [lane + (h * _L), jnp.full((_L,), t, jnp.int32)], v)

        def fire_write(i, p):
            f, cb = fc(i)
            return pltpu.async_copy(
                ob[p].at[:, pl.ds(0, 128)],
                out_hbm.at[f, :, pl.ds(cb * 128, 128)], wsem[p])

        stage_idx(0, 0).wait()
        fire_gather(0)

        def body(j2, carry):
            for p in range(2):
                i = j2 * 2 + p
                q = 1 - p
                @pl.when(i + 1 < ITERS)
                def _():
                    stage_idx(i + 1, q)
                pltpu.make_async_copy(w_hbm.at[idx[p]], gb[p],
                                      gsem[p]).wait()

                @pl.when(i + 1 < ITERS)
                def _():
                    pltpu.make_async_copy(
                        xT_hbm.at[0, pl.ds(0, 128)], idx[q],
                        isem[q]).wait()
                    fire_gather(q)

                @pl.when(i >= 2)
                def _():
                    f2, cb2 = fc(i - 2)
                    pltpu.make_async_copy(
                        ob[p].at[:, pl.ds(0, 128)],
                        out_hbm.at[f2, :, pl.ds(cb2 * 128, 128)],
                        wsem[p]).wait()
                transpose(p)
                fire_write(i, p)
            return carry

        lax.fori_loop(0, ITERS // 2, body, 0)

        f1, cb1 = fc(ITERS - 1)
        pltpu.make_async_copy(
            ob[(ITERS - 1) % 2].at[:, pl.ds(0, 128)],
            out_hbm.at[f1, :, pl.ds(cb1 * 128, 128)],
            wsem[(ITERS - 1) % 2]).wait()
        f2, cb2 = fc(ITERS - 2)
        pltpu.make_async_copy(
            ob[(ITERS - 2) % 2].at[:, pl.ds(0, 128)],
            out_hbm.at[f2, :, pl.ds(cb2 * 128, 128)],
            wsem[(ITERS - 2) % 2]).wait()

    return emb_kernel


def kernel(x, weight):
    B_TOK, F = x.shape
    V, D = weight.shape
    xT = x.astype(jnp.int32).T
    w128 = jnp.pad(weight, ((0, 0), (0, 128 - D)))
    out3 = _build(F, B_TOK, V, D)(xT, w128)
    return jnp.transpose(out3, (2, 0, 1))
